# baseline (speedup 1.0000x reference)
import functools

import jax
import jax.numpy as jnp
import numpy as np
from jax import lax
from jax.experimental import pallas as pl
from jax.experimental.pallas import tpu as pltpu
from jax.experimental.pallas import tpu_sc as plsc

_BATCH = 4
_RADIUS = 240
_ANGLE = 180
_HEIGHT = 32
_RKEEP = 120
_NPTS = 480000
_CUT = 8

_NTILES = 32
_PTS_PER_TILE = _NPTS // _NTILES
_CHUNK0 = 7504
_CHUNK1 = _PTS_PER_TILE - _CHUNK0
_AREA_THRESH = (22, 45, 67, 90, 112, 135, 157)

_PERMS = np.array([
    [1, 3, 0, 2], [2, 0, 3, 1], [0, 1, 2, 3], [3, 2, 0, 1],
    [1, 3, 2, 0], [3, 1, 2, 0], [1, 0, 3, 2], [0, 2, 1, 3],
], dtype=np.int32)


def _area_of(a):
    return sum(1 for t in _AREA_THRESH if a >= t)


def _inv_table():
    inv = np.zeros((_CUT, _BATCH), np.int32)
    for ar in range(_CUT):
        for b in range(_BATCH):
            inv[ar, _PERMS[ar, b]] = b
    return inv.reshape(-1)


def _src_map():
    m = np.zeros((_BATCH, _ANGLE), np.int32)
    for b in range(_BATCH):
        for a in range(_ANGLE):
            m[b, a] = _PERMS[_area_of(a), b]
    return m


def _sc_relabel(coords_flat, tabs):
    mesh = plsc.VectorSubcoreMesh(
        core_axis_name="c", subcore_axis_name="s", num_cores=2, num_subcores=16)

    @functools.partial(
        pl.kernel,
        out_type=jax.ShapeDtypeStruct((_NPTS * 3,), jnp.int32),
        mesh=mesh,
        compiler_params=pltpu.CompilerParams(needs_layout_passes=False),
        scratch_types=(
            pltpu.VMEM((32,), jnp.int32),
            pltpu.VMEM((_CHUNK0 * 3,), jnp.int32),
        ),
    )
    def body(coords_hbm, tabs_hbm, out_coords, tbuf, cbuf):
        wid = lax.axis_index("s") * 2 + lax.axis_index("c")
        pltpu.sync_copy(tabs_hbm, tbuf)

        cbase = wid * (_PTS_PER_TILE * 3)
        iota = lax.iota(jnp.int32, 16)

        def relabel(c0, c1, c2):
            area = (c2 >= _AREA_THRESH[0]).astype(jnp.int32)
            for t in _AREA_THRESH[1:]:
                area = area + (c2 >= t).astype(jnp.int32)
            lut = plsc.load_gather(tbuf, [area * 4 + c0])
            return jnp.where(c1 >= _RKEEP, lut, c0)

        def process(npts, off, nfull, rem):
            pltpu.sync_copy(coords_hbm.at[pl.ds(cbase + off, npts * 3)],
                            cbuf.at[pl.ds(0, npts * 3)])

            @pl.loop(0, nfull)
            def _(g):
                idx = g * 48 + iota * 3
                c0 = plsc.load_gather(cbuf, [idx])
                c1 = plsc.load_gather(cbuf, [idx + 1])
                c2 = plsc.load_gather(cbuf, [idx + 2])
                plsc.store_scatter(cbuf, [idx], relabel(c0, c1, c2))

            if rem:
                mask = iota < rem
                idx = jnp.where(mask, nfull * 48 + iota * 3, 0)
                c0 = plsc.load_gather(cbuf, [idx], mask=mask)
                c1 = plsc.load_gather(cbuf, [idx + 1], mask=mask)
                c2 = plsc.load_gather(cbuf, [idx + 2], mask=mask)
                plsc.store_scatter(cbuf, [idx], relabel(c0, c1, c2), mask=mask)

            pltpu.sync_copy(cbuf.at[pl.ds(0, npts * 3)],
                            out_coords.at[pl.ds(cbase + off, npts * 3)])

        process(_CHUNK0, 0, _CHUNK0 // 16, 0)
        process(_CHUNK1, _CHUNK0 * 3, _CHUNK1 // 16, _CHUNK1 % 16)

    return body(coords_flat, tabs)


def _tc_voxel_mix(vlt, smap):
    def body(smap_ref, own_ref, perm_ref, out_ref):
        del smap_ref
        r_idx = lax.broadcasted_iota(jnp.int32, (1, 1, _HEIGHT, _RADIUS), 3)
        out_ref[...] = jnp.where(r_idx >= _RKEEP, perm_ref[...], own_ref[...])

    grid_spec = pltpu.PrefetchScalarGridSpec(
        num_scalar_prefetch=1,
        grid=(_BATCH, _ANGLE),
        in_specs=[
            pl.BlockSpec((1, 1, _HEIGHT, _RADIUS), lambda b, a, smap: (b, a, 0, 0)),
            pl.BlockSpec((1, 1, _HEIGHT, _RADIUS),
                         lambda b, a, smap: (smap[b, a], a, 0, 0)),
        ],
        out_specs=pl.BlockSpec((1, 1, _HEIGHT, _RADIUS),
                               lambda b, a, smap: (b, a, 0, 0)),
    )
    return pl.pallas_call(
        body,
        grid_spec=grid_spec,
        out_shape=jax.ShapeDtypeStruct((_BATCH, _ANGLE, _HEIGHT, _RADIUS),
                                       jnp.int32),
    )(smap, vlt, vlt)


@jax.jit
def _impl(point_feature_, point_coord_, voxel_label_):
    tabs = jnp.asarray(_inv_table(), dtype=jnp.int32)
    smap = jnp.asarray(_src_map(), dtype=jnp.int32)
    out_c = _sc_relabel(point_coord_.reshape(-1), tabs)
    vlt = jnp.transpose(voxel_label_, (0, 2, 3, 1))
    out_t = _tc_voxel_mix(vlt, smap)
    out_v = jnp.transpose(out_t, (0, 3, 1, 2))
    return (point_feature_, out_c.reshape(_NPTS, 3), out_v)


def kernel(point_feature_, point_coord_, voxel_label_):
    return _impl(point_feature_, point_coord_, voxel_label_)


# speedup vs baseline: 5.5499x; 5.5499x over previous
import functools

import jax
import jax.numpy as jnp
import numpy as np
from jax import lax
from jax.experimental import pallas as pl
from jax.experimental.pallas import tpu as pltpu
from jax.experimental.pallas import tpu_sc as plsc

_BATCH = 4
_RADIUS = 240
_ANGLE = 180
_HEIGHT = 32
_RKEEP = 120
_NPTS = 480000
_CUT = 8

_NSUB = 32
_NT = _NPTS // 128
_BIG = 118 * 128
_SMALL = 117 * 128

_PERMS = np.array([
    [1, 3, 0, 2], [2, 0, 3, 1], [0, 1, 2, 3], [3, 2, 0, 1],
    [1, 3, 2, 0], [3, 1, 2, 0], [1, 0, 3, 2], [0, 2, 1, 3],
], dtype=np.int32)


def _area_of(a):
    return ((2 * a + 1) * 2913) >> 17


def _packed_inv():
    pk = [0] * _CUT
    for ar in range(_CUT):
        for b in range(_BATCH):
            pk[ar] |= b << (2 * int(_PERMS[ar, b]))
    return pk


_PKINV = _packed_inv()


def _src_map():
    m = np.zeros((_BATCH, _ANGLE), np.int32)
    for b in range(_BATCH):
        for a in range(_ANGLE):
            m[b, a] = _PERMS[_area_of(a), b]
    return m


def _sc_relabel(coords_t):
    mesh = plsc.VectorSubcoreMesh(
        core_axis_name="c", subcore_axis_name="s", num_cores=2, num_subcores=16)

    @functools.partial(
        pl.kernel,
        out_type=jax.ShapeDtypeStruct((3, _NPTS), jnp.int32),
        mesh=mesh,
        compiler_params=pltpu.CompilerParams(
            needs_layout_passes=False, use_tc_tiling_on_sc=True),
        scratch_types=(pltpu.VMEM((3, _BIG), jnp.int32),),
    )
    def body(coords_hbm, out_coords, buf):
        wid = lax.axis_index("s") * 2 + lax.axis_index("c")
        base = wid * _SMALL + jnp.minimum(wid, 6) * 128

        def go(npts):
            pltpu.sync_copy(coords_hbm.at[:, pl.ds(base, npts)],
                            buf.at[:, pl.ds(0, npts)])

            @pl.loop(0, npts // 16, unroll=4)
            def _(g):
                idx = g * 16
                c0 = buf[0, pl.ds(idx, 16)]
                c1 = buf[1, pl.ds(idx, 16)]
                c2 = buf[2, pl.ds(idx, 16)]
                area = ((c2 * 2 + 1) * 2913) >> 17
                pk = jnp.full((16,), _PKINV[7], jnp.int32)
                for ar in range(6, -1, -1):
                    pk = jnp.where(area == ar, _PKINV[ar], pk)
                new0 = (pk >> (c0 * 2)) & 3
                buf[0, pl.ds(idx, 16)] = jnp.where(c1 >= _RKEEP, new0, c0)

            pltpu.sync_copy(buf.at[:, pl.ds(0, npts)],
                            out_coords.at[:, pl.ds(base, npts)])

        @pl.when(wid < 6)
        def _():
            go(_BIG)

        @pl.when(wid >= 6)
        def _():
            go(_SMALL)

    return body(coords_t)


def _tc_voxel_mix(vlt, smap):
    def body(smap_ref, own_ref, perm_ref, out_ref):
        del smap_ref
        r_idx = lax.broadcasted_iota(jnp.int32, (1, 1, _HEIGHT, _RADIUS), 3)
        out_ref[...] = jnp.where(r_idx >= _RKEEP, perm_ref[...], own_ref[...])

    grid_spec = pltpu.PrefetchScalarGridSpec(
        num_scalar_prefetch=1,
        grid=(_BATCH, _ANGLE),
        in_specs=[
            pl.BlockSpec((1, 1, _HEIGHT, _RADIUS), lambda b, a, smap: (b, a, 0, 0)),
            pl.BlockSpec((1, 1, _HEIGHT, _RADIUS),
                         lambda b, a, smap: (smap[b, a], a, 0, 0)),
        ],
        out_specs=pl.BlockSpec((1, 1, _HEIGHT, _RADIUS),
                               lambda b, a, smap: (b, a, 0, 0)),
    )
    return pl.pallas_call(
        body,
        grid_spec=grid_spec,
        out_shape=jax.ShapeDtypeStruct((_BATCH, _ANGLE, _HEIGHT, _RADIUS),
                                       jnp.int32),
    )(smap, vlt, vlt)


@jax.jit
def _impl(point_feature_, point_coord_, voxel_label_):
    smap = jnp.asarray(_src_map(), dtype=jnp.int32)
    out_ct = _sc_relabel(jnp.transpose(point_coord_))
    out_c = jnp.transpose(out_ct)
    vlt = jnp.transpose(voxel_label_, (0, 2, 3, 1))
    out_t = _tc_voxel_mix(vlt, smap)
    out_v = jnp.transpose(out_t, (0, 3, 1, 2))
    return (point_feature_, out_c, out_v)


def kernel(point_feature_, point_coord_, voxel_label_):
    return _impl(point_feature_, point_coord_, voxel_label_)
